# baseline (speedup 1.0000x reference)
import functools

import jax
import jax.numpy as jnp
from jax import lax
from jax.experimental import pallas as pl
from jax.experimental.pallas import tpu as pltpu
from jax.experimental.pallas import tpu_sc as plsc


@functools.cache
def _make_gather(V, D, B, H):
    info = plsc.get_sparse_core_info()
    NC, NS = info.num_cores, info.num_subcores
    NW = NC * NS
    N = B * H
    assert N % NW == 0
    n_per_w = N // NW
    block = 512
    while n_per_w % block or B % block:
        block //= 2
    n_blocks = n_per_w // block
    q_per_h = B // block
    L = info.num_lanes
    mesh = plsc.VectorSubcoreMesh(core_axis_name="c", subcore_axis_name="s")

    @functools.partial(
        pl.kernel,
        out_type=jax.ShapeDtypeStruct((H, D, B), jnp.float32),
        mesh=mesh,
        scratch_types=[
            pltpu.VMEM((n_per_w,), jnp.int32),
            [pltpu.VMEM((block, D), jnp.float32) for _ in range(2)],
            [pltpu.VMEM((D, block), jnp.float32) for _ in range(2)],
            [pltpu.SemaphoreType.DMA for _ in range(2)],
            [pltpu.SemaphoreType.DMA for _ in range(2)],
        ],
        compiler_params=pltpu.CompilerParams(
            use_tc_tiling_on_sc=False, needs_layout_passes=False
        ),
    )
    def gather_kernel(
        table_hbm, idx_hbm, out_hbm, idx_v, bufs, bufts, sems_g, sems_w
    ):
        wid = lax.axis_index("s") * NC + lax.axis_index("c")
        base = wid * n_per_w
        first_blk = wid * n_blocks
        pltpu.sync_copy(idx_hbm.at[pl.ds(base, n_per_w)], idx_v)

        lanes = lax.iota(jnp.int32, L)

        def issue_gather(j, s):
            pltpu.async_copy(
                table_hbm.at[idx_v.at[pl.ds(j * block, block)]],
                bufs[s],
                sems_g[s],
            )

        def wait_gather(s):
            pltpu.make_async_copy(
                table_hbm.at[pl.ds(0, block)], bufs[s], sems_g[s]
            ).wait()

        row_ids = [d0 + lanes for d0 in range(0, D, L)]

        def transpose(s):
            buf, buft = bufs[s], bufts[s]

            def trow(r):
                col = jnp.full((L,), r, jnp.int32)
                for k, d0 in enumerate(range(0, D, L)):
                    v = buf[r, pl.ds(d0, L)]
                    plsc.store_scatter(buft, [row_ids[k], col], v)

            pl.loop(0, block, unroll=8)(trow)

        def issue_wb(j, s):
            blk = first_blk + j
            h = blk // q_per_h
            q = blk - h * q_per_h
            pltpu.async_copy(
                bufts[s],
                out_hbm.at[h, :, pl.ds(q * block, block)],
                sems_w[s],
            )

        def wait_wb(s):
            pltpu.make_async_copy(
                table_hbm.at[pl.ds(0, block)], bufs[s], sems_w[s]
            ).wait()

        issue_gather(0, 0)

        def outer(j0):
            for u in range(2):
                j = j0 + u
                s = u
                t = 1 - u

                @pl.when(j + 1 < n_blocks)
                def _ahead():
                    issue_gather(j + 1, t)

                wait_gather(s)

                @pl.when(j >= 2)
                def _free():
                    wait_wb(s)

                transpose(s)
                issue_wb(j, s)

        pl.loop(0, n_blocks, step=2)(outer)
        wait_wb(0)
        wait_wb(1)

    return gather_kernel


def kernel(embed_id, weight):
    bsz, hist = embed_id.shape
    V, D = weight.shape
    flat_idx = jnp.transpose(embed_id).reshape(bsz * hist).astype(jnp.int32)
    out = _make_gather(V, D, bsz, hist)(weight, flat_idx)
    return jnp.transpose(out, (2, 0, 1))


# speedup vs baseline: 1.0849x; 1.0849x over previous
import functools

import jax
import jax.numpy as jnp
from jax import lax
from jax.experimental import pallas as pl
from jax.experimental.pallas import tpu as pltpu
from jax.experimental.pallas import tpu_sc as plsc


@functools.cache
def _make_gather(V, D, B, H):
    info = plsc.get_sparse_core_info()
    NC, NS = info.num_cores, info.num_subcores
    NW = NC * NS
    N = B * H
    assert N % NW == 0
    n_per_w = N // NW
    block = 512
    while n_per_w % block or B % block:
        block //= 2
    n_blocks = n_per_w // block
    q_per_h = B // block
    L = info.num_lanes
    mesh = plsc.VectorSubcoreMesh(core_axis_name="c", subcore_axis_name="s")

    @functools.partial(
        pl.kernel,
        out_type=jax.ShapeDtypeStruct((H, D // 8, B // 128, 8, 128), jnp.float32),
        mesh=mesh,
        scratch_types=[
            pltpu.VMEM((n_per_w,), jnp.int32),
            [pltpu.VMEM((block, D), jnp.float32) for _ in range(2)],
            [pltpu.VMEM((D, block), jnp.float32) for _ in range(2)],
            [pltpu.SemaphoreType.DMA for _ in range(2)],
            [pltpu.SemaphoreType.DMA for _ in range(2)],
        ],
        compiler_params=pltpu.CompilerParams(
            use_tc_tiling_on_sc=False, needs_layout_passes=False
        ),
    )
    def gather_kernel(
        table_hbm, idx_hbm, out_hbm, idx_v, bufs, bufts, sems_g, sems_w
    ):
        wid = lax.axis_index("s") * NC + lax.axis_index("c")
        base = wid * n_per_w
        first_blk = wid * n_blocks
        pltpu.sync_copy(idx_hbm.at[pl.ds(base, n_per_w)], idx_v)

        lanes = lax.iota(jnp.int32, L)

        def issue_gather(j, s):
            pltpu.async_copy(
                table_hbm.at[idx_v.at[pl.ds(j * block, block)]],
                bufs[s],
                sems_g[s],
            )

        def wait_gather(s):
            pltpu.make_async_copy(
                table_hbm.at[pl.ds(0, block)], bufs[s], sems_g[s]
            ).wait()

        row_ids = [d0 + lanes for d0 in range(0, D, L)]

        def transpose(s):
            buf, buft = bufs[s], bufts[s]

            def trow(r):
                col = jnp.full((L,), r, jnp.int32)
                for k, d0 in enumerate(range(0, D, L)):
                    v = buf[r, pl.ds(d0, L)]
                    plsc.store_scatter(buft, [row_ids[k], col], v)

            pl.loop(0, block, unroll=8)(trow)

        def issue_wb(j, s):
            blk = first_blk + j
            h = blk // q_per_h
            q = blk - h * q_per_h
            c0 = q * (block // 128)
            for g in range(D // 8):
                for cl in range(block // 128):
                    pltpu.async_copy(
                        bufts[s].at[pl.ds(g * 8, 8), pl.ds(cl * 128, 128)],
                        out_hbm.at[h, g, c0 + cl],
                        sems_w[s],
                    )

        def wait_wb(s):
            pltpu.make_async_copy(
                table_hbm.at[pl.ds(0, block)], bufs[s], sems_w[s]
            ).wait()

        issue_gather(0, 0)

        def outer(j0):
            for u in range(2):
                j = j0 + u
                s = u
                t = 1 - u

                @pl.when(j + 1 < n_blocks)
                def _ahead():
                    issue_gather(j + 1, t)

                wait_gather(s)

                @pl.when(j >= 2)
                def _free():
                    wait_wb(s)

                transpose(s)
                issue_wb(j, s)

        pl.loop(0, n_blocks, step=2)(outer)
        wait_wb(0)
        wait_wb(1)

    return gather_kernel


def kernel(embed_id, weight):
    bsz, hist = embed_id.shape
    V, D = weight.shape
    flat_idx = jnp.transpose(embed_id).reshape(bsz * hist).astype(jnp.int32)
    out5 = _make_gather(V, D, bsz, hist)(weight, flat_idx)
    return jnp.transpose(out5, (2, 4, 0, 1, 3)).reshape(bsz, hist, D)


# speedup vs baseline: 1.2636x; 1.1647x over previous
import functools

import jax
import jax.numpy as jnp
from jax import lax
from jax.experimental import pallas as pl
from jax.experimental.pallas import tpu as pltpu
from jax.experimental.pallas import tpu_sc as plsc


@functools.cache
def _make_gather(V, D, B, H):
    info = plsc.get_sparse_core_info()
    NC, NS = info.num_cores, info.num_subcores
    NW = NC * NS
    N = B * H
    assert N % NW == 0
    n_per_w = N // NW
    block = 512
    while n_per_w % block or B % block:
        block //= 2
    n_blocks = n_per_w // block
    q_per_h = B // block
    L = info.num_lanes
    mesh = plsc.VectorSubcoreMesh(core_axis_name="c", subcore_axis_name="s")

    @functools.partial(
        pl.kernel,
        out_type=jax.ShapeDtypeStruct((H, D // 8, B // 128, 8, 128), jnp.float32),
        mesh=mesh,
        scratch_types=[
            pltpu.VMEM((n_per_w,), jnp.int32),
            [pltpu.VMEM((block, D), jnp.float32) for _ in range(2)],
            [pltpu.VMEM((D, block), jnp.float32) for _ in range(2)],
            [pltpu.SemaphoreType.DMA for _ in range(2)],
            [pltpu.SemaphoreType.DMA for _ in range(2)],
        ],
        compiler_params=pltpu.CompilerParams(
            use_tc_tiling_on_sc=False, needs_layout_passes=False
        ),
    )
    def gather_kernel(
        table_hbm, idx_hbm, out_hbm, idx_v, bufs, bufts, sems_g, sems_w
    ):
        wid = lax.axis_index("s") * NC + lax.axis_index("c")
        base = wid * n_per_w
        first_blk = wid * n_blocks
        pltpu.sync_copy(idx_hbm.at[pl.ds(base, n_per_w)], idx_v)

        lanes = lax.iota(jnp.int32, L)

        def issue_gather(j, s):
            pltpu.async_copy(
                table_hbm.at[idx_v.at[pl.ds(j * block, block)]],
                bufs[s],
                sems_g[s],
            )

        def wait_gather(s):
            pltpu.make_async_copy(
                table_hbm.at[pl.ds(0, block)], bufs[s], sems_g[s]
            ).wait()

        row_ids = [d0 + lanes for d0 in range(0, D, L)]

        def transpose(s):
            buf, buft = bufs[s], bufts[s]

            def trow(r):
                col = jnp.full((L,), r, jnp.int32)
                for k, d0 in enumerate(range(0, D, L)):
                    v = buf[r, pl.ds(d0, L)]
                    plsc.store_scatter(buft, [row_ids[k], col], v)

            plsc.parallel_loop(0, block, unroll=8)(trow)

        def issue_wb(j, s):
            blk = first_blk + j
            h = blk // q_per_h
            q = blk - h * q_per_h
            c0 = q * (block // 128)
            for g in range(D // 8):
                for cl in range(block // 128):
                    pltpu.async_copy(
                        bufts[s].at[pl.ds(g * 8, 8), pl.ds(cl * 128, 128)],
                        out_hbm.at[h, g, c0 + cl],
                        sems_w[s],
                    )

        def wait_wb(s):
            pltpu.make_async_copy(
                table_hbm.at[pl.ds(0, block)], bufs[s], sems_w[s]
            ).wait()

        issue_gather(0, 0)

        def outer(j0):
            for u in range(2):
                j = j0 + u
                s = u
                t = 1 - u

                @pl.when(j + 1 < n_blocks)
                def _ahead():
                    issue_gather(j + 1, t)

                wait_gather(s)

                @pl.when(j >= 2)
                def _free():
                    wait_wb(s)

                transpose(s)
                issue_wb(j, s)

        pl.loop(0, n_blocks, step=2)(outer)
        wait_wb(0)
        wait_wb(1)

    return gather_kernel


def kernel(embed_id, weight):
    bsz, hist = embed_id.shape
    V, D = weight.shape
    flat_idx = jnp.transpose(embed_id).reshape(bsz * hist).astype(jnp.int32)
    out5 = _make_gather(V, D, bsz, hist)(weight, flat_idx)
    return jnp.transpose(out5, (2, 4, 0, 1, 3)).reshape(bsz, hist, D)


# speedup vs baseline: 1.2652x; 1.0013x over previous
import functools

import jax
import jax.numpy as jnp
from jax import lax
from jax.experimental import pallas as pl
from jax.experimental.pallas import tpu as pltpu
from jax.experimental.pallas import tpu_sc as plsc


@functools.cache
def _make_gather(V, D, B, H):
    info = plsc.get_sparse_core_info()
    NC, NS = info.num_cores, info.num_subcores
    NW = NC * NS
    N = B * H
    assert N % NW == 0
    n_per_w = N // NW
    block = 512
    while n_per_w % block or B % block:
        block //= 2
    n_blocks = n_per_w // block
    q_per_h = B // block
    L = info.num_lanes
    mesh = plsc.VectorSubcoreMesh(core_axis_name="c", subcore_axis_name="s")

    @functools.partial(
        pl.kernel,
        out_type=jax.ShapeDtypeStruct((H, D // 8, B // 128, 8, 128), jnp.float32),
        mesh=mesh,
        scratch_types=[
            pltpu.VMEM((n_per_w,), jnp.int32),
            [pltpu.VMEM((block, D), jnp.float32) for _ in range(2)],
            [pltpu.VMEM((D, block), jnp.float32) for _ in range(2)],
            [pltpu.SemaphoreType.DMA for _ in range(2)],
            [pltpu.SemaphoreType.DMA for _ in range(2)],
        ],
        compiler_params=pltpu.CompilerParams(
            use_tc_tiling_on_sc=False, needs_layout_passes=False
        ),
    )
    def gather_kernel(
        table_hbm, idx_hbm, out_hbm, idx_v, bufs, bufts, sems_g, sems_w
    ):
        wid = lax.axis_index("s") * NC + lax.axis_index("c")
        base = wid * n_per_w
        first_blk = wid * n_blocks
        pltpu.sync_copy(idx_hbm.at[pl.ds(base, n_per_w)], idx_v)

        lanes = lax.iota(jnp.int32, L)

        def issue_gather(j, s):
            pltpu.async_copy(
                table_hbm.at[idx_v.at[pl.ds(j * block, block)]],
                bufs[s],
                sems_g[s],
            )

        def wait_gather(s):
            pltpu.make_async_copy(
                table_hbm.at[pl.ds(0, block)], bufs[s], sems_g[s]
            ).wait()

        row_ids = [d0 + lanes for d0 in range(0, D, L)]

        def transpose(s):
            buf, buft = bufs[s], bufts[s]

            def trow(r):
                col = jnp.full((L,), r, jnp.int32)
                for k, d0 in enumerate(range(0, D, L)):
                    v = buf[r, pl.ds(d0, L)]
                    plsc.store_scatter(buft, [row_ids[k], col], v)

            plsc.parallel_loop(0, block, unroll=16)(trow)

        def issue_wb(j, s):
            blk = first_blk + j
            h = blk // q_per_h
            q = blk - h * q_per_h
            c0 = q * (block // 128)
            for g in range(D // 8):
                for cl in range(block // 128):
                    pltpu.async_copy(
                        bufts[s].at[pl.ds(g * 8, 8), pl.ds(cl * 128, 128)],
                        out_hbm.at[h, g, c0 + cl],
                        sems_w[s],
                    )

        def wait_wb(s):
            pltpu.make_async_copy(
                table_hbm.at[pl.ds(0, block)], bufs[s], sems_w[s]
            ).wait()

        issue_gather(0, 0)

        def outer(j0):
            for u in range(2):
                j = j0 + u
                s = u
                t = 1 - u

                @pl.when(j + 1 < n_blocks)
                def _ahead():
                    issue_gather(j + 1, t)

                wait_gather(s)

                @pl.when(j >= 2)
                def _free():
                    wait_wb(s)

                transpose(s)
                issue_wb(j, s)

        pl.loop(0, n_blocks, step=2)(outer)
        wait_wb(0)
        wait_wb(1)

    return gather_kernel


def kernel(embed_id, weight):
    bsz, hist = embed_id.shape
    V, D = weight.shape
    flat_idx = jnp.transpose(embed_id).reshape(bsz * hist).astype(jnp.int32)
    out5 = _make_gather(V, D, bsz, hist)(weight, flat_idx)
    return jnp.transpose(out5, (2, 4, 0, 1, 3)).reshape(bsz, hist, D)


# speedup vs baseline: 1.3700x; 1.0829x over previous
import functools

import jax
import jax.numpy as jnp
from jax import lax
from jax.experimental import pallas as pl
from jax.experimental.pallas import tpu as pltpu
from jax.experimental.pallas import tpu_sc as plsc


@functools.cache
def _make_transpose(V, D):
    info = plsc.get_sparse_core_info()
    NC, NS, L = info.num_cores, info.num_subcores, info.num_lanes
    NW = NC * NS
    CW = 512
    n_main = (V // CW) * CW
    n_chunks = n_main // CW
    tail_n = V - n_main
    per_w = -(-n_chunks // NW)
    mesh = plsc.VectorSubcoreMesh(core_axis_name="c", subcore_axis_name="s")

    @functools.partial(
        pl.kernel,
        out_type=jax.ShapeDtypeStruct((V * D,), jnp.float32),
        mesh=mesh,
        scratch_types=[
            [pltpu.VMEM((D, CW), jnp.float32) for _ in range(2)],
            [pltpu.VMEM((CW * D,), jnp.float32) for _ in range(2)],
            pltpu.VMEM((tail_n, D), jnp.float32),
            [pltpu.SemaphoreType.DMA for _ in range(2)],
            [pltpu.SemaphoreType.DMA for _ in range(2)],
        ],
        compiler_params=pltpu.CompilerParams(
            use_tc_tiling_on_sc=True, needs_layout_passes=False
        ),
    )
    def transpose_kernel(wt_hbm, tail_hbm, out_hbm, bins, bouts, btail, sems_i, sems_o):
        wid = lax.axis_index("s") * NC + lax.axis_index("c")

        @pl.when(wid == 0)
        def _tail():
            pltpu.sync_copy(tail_hbm, btail)
            for r in range(tail_n):
                pltpu.sync_copy(
                    btail.at[r], out_hbm.at[pl.ds((n_main + r) * D, D)]
                )

        lanes = lax.iota(jnp.int32, L)
        row_ids = [d0 + lanes for d0 in range(0, D, L)]

        def chunk_of(k):
            return wid + k * NW

        def issue_in(k, s):
            c = chunk_of(k)
            pltpu.async_copy(
                wt_hbm.at[:, pl.ds(c * CW, CW)], bins[s], sems_i[s]
            )

        def wait_in(s):
            pltpu.make_async_copy(
                wt_hbm.at[:, pl.ds(0, CW)], bins[s], sems_i[s]
            ).wait()

        def transpose(s):
            bi, bo = bins[s], bouts[s]

            def trow(r):
                col = jnp.full((L,), r, jnp.int32)
                for i, d0 in enumerate(range(0, D, L)):
                    v = plsc.load_gather(bi, [row_ids[i], col])
                    bo[pl.ds(r * D + d0, L)] = v

            plsc.parallel_loop(0, CW, unroll=8)(trow)

        def issue_out(k, s):
            c = chunk_of(k)
            pltpu.async_copy(
                bouts[s],
                out_hbm.at[pl.ds(c * CW * D, CW * D)],
                sems_o[s],
            )

        def wait_out(s):
            pltpu.make_async_copy(
                out_hbm.at[pl.ds(0, CW * D)], bouts[s], sems_o[s]
            ).wait()

        @pl.when(chunk_of(0) < n_chunks)
        def _p():
            issue_in(0, 0)

        def body(k):
            for u in range(2):
                kk = k + u
                s = u
                t = 1 - u

                @pl.when(chunk_of(kk) < n_chunks)
                def _work():
                    @pl.when(chunk_of(kk + 1) < n_chunks)
                    def _ahead():
                        issue_in(kk + 1, t)

                    wait_in(s)

                    @pl.when(kk >= 2)
                    def _free():
                        wait_out(s)

                    transpose(s)
                    issue_out(kk, s)

        pl.loop(0, per_w + (per_w % 2), step=2)(body)

        @pl.when(chunk_of(per_w - 1) < n_chunks)
        def _d1():
            wait_out((per_w - 1) % 2)

        @pl.when(chunk_of(per_w - 2) < n_chunks)
        def _d2():
            wait_out((per_w - 2) % 2)

        @pl.when(
            (chunk_of(per_w - 1) >= n_chunks) & (chunk_of(per_w - 3) < n_chunks)
        )
        def _d3():
            wait_out((per_w - 3) % 2)

    return transpose_kernel


@functools.cache
def _make_gather(V, D, B, H):
    info = plsc.get_sparse_core_info()
    NC, NS = info.num_cores, info.num_subcores
    NW = NC * NS
    N = B * H
    assert N % NW == 0
    n_per_w = N // NW
    block = 512
    while n_per_w % block or B % block:
        block //= 2
    n_blocks = n_per_w // block
    q_per_h = B // block
    L = info.num_lanes
    mesh = plsc.VectorSubcoreMesh(core_axis_name="c", subcore_axis_name="s")

    @functools.partial(
        pl.kernel,
        out_type=jax.ShapeDtypeStruct((H, D // 8, B // 128, 8, 128), jnp.float32),
        mesh=mesh,
        scratch_types=[
            pltpu.VMEM((n_per_w,), jnp.int32),
            [pltpu.VMEM((block, D), jnp.float32) for _ in range(2)],
            [pltpu.VMEM((D, block), jnp.float32) for _ in range(2)],
            [pltpu.SemaphoreType.DMA for _ in range(2)],
            [pltpu.SemaphoreType.DMA for _ in range(2)],
        ],
        compiler_params=pltpu.CompilerParams(
            use_tc_tiling_on_sc=False, needs_layout_passes=False
        ),
    )
    def gather_kernel(
        table_hbm, idx_hbm, out_hbm, idx_v, bufs, bufts, sems_g, sems_w
    ):
        wid = lax.axis_index("s") * NC + lax.axis_index("c")
        base = wid * n_per_w
        first_blk = wid * n_blocks
        pltpu.sync_copy(idx_hbm.at[pl.ds(base, n_per_w)], idx_v)

        lanes = lax.iota(jnp.int32, L)

        def issue_gather(j, s):
            pltpu.async_copy(
                table_hbm.at[idx_v.at[pl.ds(j * block, block)]],
                bufs[s],
                sems_g[s],
            )

        def wait_gather(s):
            pltpu.make_async_copy(
                table_hbm.at[pl.ds(0, block)], bufs[s], sems_g[s]
            ).wait()

        row_ids = [d0 + lanes for d0 in range(0, D, L)]

        def transpose(s):
            buf, buft = bufs[s], bufts[s]

            def trow(r):
                col = jnp.full((L,), r, jnp.int32)
                for k, d0 in enumerate(range(0, D, L)):
                    v = buf[r, pl.ds(d0, L)]
                    plsc.store_scatter(buft, [row_ids[k], col], v)

            plsc.parallel_loop(0, block, unroll=8)(trow)

        def issue_wb(j, s):
            blk = first_blk + j
            h = blk // q_per_h
            q = blk - h * q_per_h
            c0 = q * (block // 128)
            for g in range(D // 8):
                for cl in range(block // 128):
                    pltpu.async_copy(
                        bufts[s].at[pl.ds(g * 8, 8), pl.ds(cl * 128, 128)],
                        out_hbm.at[h, g, c0 + cl],
                        sems_w[s],
                    )

        def wait_wb(s):
            pltpu.make_async_copy(
                table_hbm.at[pl.ds(0, block)], bufs[s], sems_w[s]
            ).wait()

        issue_gather(0, 0)

        def outer(j0):
            for u in range(2):
                j = j0 + u
                s = u
                t = 1 - u

                @pl.when(j + 1 < n_blocks)
                def _ahead():
                    issue_gather(j + 1, t)

                wait_gather(s)

                @pl.when(j >= 2)
                def _free():
                    wait_wb(s)

                transpose(s)
                issue_wb(j, s)

        pl.loop(0, n_blocks, step=2)(outer)
        wait_wb(0)
        wait_wb(1)

    return gather_kernel


def kernel(embed_id, weight):
    bsz, hist = embed_id.shape
    V, D = weight.shape
    n_main = (V // 512) * 512
    wT = jnp.transpose(weight)
    tail = weight[n_main:]
    wlin = _make_transpose(V, D)(wT, tail)
    table = wlin.reshape(V, D)
    flat_idx = jnp.transpose(embed_id).reshape(bsz * hist).astype(jnp.int32)
    out5 = _make_gather(V, D, bsz, hist)(table, flat_idx)
    return jnp.transpose(out5, (2, 4, 0, 1, 3)).reshape(bsz, hist, D)
